# Initial kernel scaffold; baseline (speedup 1.0000x reference)
import jax
import jax.numpy as jnp
from jax.experimental import pallas as pl


def kernel(src_tokens, trg_tokens, src_emb, trg_emb, W_out, b_out):
    raise NotImplementedError("write your pallas kernel here")


# speedup vs baseline: 11.1812x; 11.1812x over previous
import jax
import jax.numpy as jnp
from jax import lax
from jax.experimental import pallas as pl
from jax.experimental.pallas import tpu as pltpu
from jax.experimental.pallas import tpu_sc as plsc

PAD = 1
B, L, D = 4096, 50, 64
NC, NS = 2, 16
NW = NC * NS
BPW = B // NW
NBUF = 5
VL = 16


def _body(idx_s_hbm, idx_t_hbm, semb_hbm, temb_hbm, par_hbm, out_hbm,
          idxs_v, idxt_v, sbuf, tbuf, w_v, recs_v, rect_v, dots_v, dott_v,
          out_v, ssems, tsems):
    wid = lax.axis_index("s") * NC + lax.axis_index("c")
    pltpu.sync_copy(idx_s_hbm.at[wid], idxs_v)
    pltpu.sync_copy(idx_t_hbm.at[wid], idxt_v)
    pltpu.sync_copy(par_hbm, w_v)

    def fire_all(emb, idx_v, buf, sems):
        for b in range(NBUF):
            pltpu.make_async_copy(emb.at[idx_v.at[b]], buf.at[b],
                                  sems.at[b]).start()

        def loop(i, _):
            l0 = i * NBUF
            for b in range(NBUF):
                pltpu.make_async_copy(emb.at[idx_v.at[0]], buf.at[b],
                                      sems.at[b]).wait()
                pltpu.make_async_copy(emb.at[idx_v.at[l0 + b]], buf.at[b],
                                      sems.at[b]).start(add=True)
            return 0

        lax.fori_loop(1, L // NBUF, loop, 0)

    fire_all(semb_hbm, idxs_v, sbuf, ssems)
    fire_all(temb_hbm, idxt_v, tbuf, tsems)

    def counts(idx_v, rec_v):
        for c in range(BPW // VL):
            def cbody(l, acc):
                t = idx_v[l, pl.ds(c * VL, VL)]
                return acc + jnp.where(t != PAD, 1, 0).astype(jnp.int32)
            cnt = lax.fori_loop(0, L, cbody, jnp.zeros((VL,), jnp.int32))
            rec_v[pl.ds(c * VL, VL)] = 1.0 / cnt.astype(jnp.float32)

    counts(idxs_v, recs_v)
    counts(idxt_v, rect_v)

    def drain(emb, idx_v, buf, sems):
        for b in range(NBUF):
            pltpu.make_async_copy(emb.at[idx_v.at[0]], buf.at[b],
                                  sems.at[b]).wait()

    def dots(buf, wofs, dot_v):
        wch = [w_v[pl.ds(wofs + c * VL, VL)] for c in range(D // VL)]
        lanes = lax.iota(jnp.int32, VL)

        def chunk(ci, _):
            dotvec = jnp.zeros((VL,), jnp.float32)
            for j in range(VL):
                r = ci * VL + j
                acc = jnp.zeros((VL,), jnp.float32)
                for nb in range(NBUF):
                    for c in range(D // VL):
                        acc = acc + buf[nb, r, pl.ds(c * VL, VL)] * wch[c]
                dotvec = jnp.where(lanes == j, jnp.sum(acc), dotvec)
            dot_v[pl.ds(ci * VL, VL)] = dotvec
            return 0

        lax.fori_loop(0, BPW // VL, chunk, 0)

    drain(semb_hbm, idxs_v, sbuf, ssems)
    dots(sbuf, 0, dots_v)
    drain(temb_hbm, idxt_v, tbuf, tsems)
    dots(tbuf, D, dott_v)

    b0 = w_v[pl.ds(2 * D, VL)][0]
    for c in range(BPW // VL):
        sl = pl.ds(c * VL, VL)
        s = dots_v[sl] * recs_v[sl] + dott_v[sl] * rect_v[sl] + b0
        out_v[sl] = 1.0 / (1.0 + jnp.exp(-s))

    pltpu.sync_copy(out_v, out_hbm.at[pl.ds(wid * BPW, BPW)])


_sc_call_cache = []


def _get_sc_call():
    if not _sc_call_cache:
        mesh = plsc.VectorSubcoreMesh(core_axis_name="c", subcore_axis_name="s",
                                      num_cores=NC, num_subcores=NS)
        _sc_call_cache.append(pl.kernel(
            _body,
            out_type=jax.ShapeDtypeStruct((B,), jnp.float32),
            mesh=mesh,
            compiler_params=pltpu.CompilerParams(needs_layout_passes=False,
                                                 use_tc_tiling_on_sc=False),
            scratch_types=[
                pltpu.VMEM((L, BPW), jnp.int32),
                pltpu.VMEM((L, BPW), jnp.int32),
                pltpu.VMEM((NBUF, BPW, D), jnp.float32),
                pltpu.VMEM((NBUF, BPW, D), jnp.float32),
                pltpu.VMEM((2 * D + VL,), jnp.float32),
                pltpu.VMEM((BPW,), jnp.float32),
                pltpu.VMEM((BPW,), jnp.float32),
                pltpu.VMEM((BPW,), jnp.float32),
                pltpu.VMEM((BPW,), jnp.float32),
                pltpu.VMEM((BPW,), jnp.float32),
                pltpu.SemaphoreType.DMA((NBUF,)),
                pltpu.SemaphoreType.DMA((NBUF,)),
            ],
        ))
    return _sc_call_cache[0]


@jax.jit
def kernel(src_tokens, trg_tokens, src_emb, trg_emb, W_out, b_out):
    idx_s = src_tokens.astype(jnp.int32).reshape(NW, BPW, L).transpose(0, 2, 1)
    idx_t = trg_tokens.astype(jnp.int32).reshape(NW, BPW, L).transpose(0, 2, 1)
    par = jnp.concatenate([W_out.reshape(-1), b_out.reshape(-1),
                           jnp.zeros((VL - 1,), jnp.float32)])
    out = _get_sc_call()(idx_s, idx_t, src_emb, trg_emb, par)
    return out.reshape(B, 1)


# speedup vs baseline: 11.2315x; 1.0045x over previous
import jax
import jax.numpy as jnp
from jax import lax
from jax.experimental import pallas as pl
from jax.experimental.pallas import tpu as pltpu
from jax.experimental.pallas import tpu_sc as plsc

PAD = 1
B, L, D = 4096, 50, 64
NC, NS = 2, 16
NW = NC * NS
BPW = B // NW
NBUF = 5
VL = 16


def _body(tok_s_hbm, tok_t_hbm, semb_hbm, temb_hbm, par_hbm, out_hbm,
          toks_v, tokt_v, idxs_v, idxt_v, sbuf, tbuf, w_v,
          recs_v, rect_v, dots_v, dott_v, out_v, ssems, tsems):
    wid = lax.axis_index("s") * NC + lax.axis_index("c")
    pltpu.sync_copy(tok_s_hbm.at[pl.ds(wid * BPW, BPW)], toks_v)
    pltpu.sync_copy(tok_t_hbm.at[pl.ds(wid * BPW, BPW)], tokt_v)
    pltpu.sync_copy(par_hbm, w_v)

    rows16 = lax.iota(jnp.int32, VL)

    def transpose(tok_v, idx_v):
        for c in range(BPW // VL):
            base = rows16 + c * VL

            def tbody(l, _):
                col = jnp.full((VL,), l, jnp.int32)
                idx_v[l, pl.ds(c * VL, VL)] = plsc.load_gather(
                    tok_v, [base, col])
                return 0

            lax.fori_loop(0, L, tbody, 0)

    transpose(toks_v, idxs_v)
    transpose(tokt_v, idxt_v)

    for b in range(NBUF):
        pltpu.make_async_copy(semb_hbm.at[idxs_v.at[b]], sbuf.at[b],
                              ssems.at[b]).start()
        pltpu.make_async_copy(temb_hbm.at[idxt_v.at[b]], tbuf.at[b],
                              tsems.at[b]).start()

    def loop(i, _):
        l0 = i * NBUF
        for b in range(NBUF):
            pltpu.make_async_copy(semb_hbm.at[idxs_v.at[0]], sbuf.at[b],
                                  ssems.at[b]).wait()
            pltpu.make_async_copy(semb_hbm.at[idxs_v.at[l0 + b]], sbuf.at[b],
                                  ssems.at[b]).start(add=True)
            pltpu.make_async_copy(temb_hbm.at[idxt_v.at[0]], tbuf.at[b],
                                  tsems.at[b]).wait()
            pltpu.make_async_copy(temb_hbm.at[idxt_v.at[l0 + b]], tbuf.at[b],
                                  tsems.at[b]).start(add=True)
        return 0

    lax.fori_loop(1, L // NBUF, loop, 0)

    def counts(idx_v, rec_v):
        for c in range(BPW // VL):
            def cbody(l, acc):
                t = idx_v[l, pl.ds(c * VL, VL)]
                return acc + jnp.where(t != PAD, 1, 0).astype(jnp.int32)
            cnt = lax.fori_loop(0, L, cbody, jnp.zeros((VL,), jnp.int32))
            rec_v[pl.ds(c * VL, VL)] = 1.0 / cnt.astype(jnp.float32)

    counts(idxs_v, recs_v)
    counts(idxt_v, rect_v)

    def drain(emb, idx_v, buf, sems):
        for b in range(NBUF):
            pltpu.make_async_copy(emb.at[idx_v.at[0]], buf.at[b],
                                  sems.at[b]).wait()

    def dots(buf, wofs, dot_v):
        wch = [w_v[0, pl.ds(wofs + c * VL, VL)] for c in range(D // VL)]
        lanes = lax.iota(jnp.int32, VL)

        def chunk(ci, _):
            dotvec = jnp.zeros((VL,), jnp.float32)
            for j in range(VL):
                r = ci * VL + j
                acc = jnp.zeros((VL,), jnp.float32)
                for nb in range(NBUF):
                    for c in range(D // VL):
                        acc = acc + buf[nb, r, pl.ds(c * VL, VL)] * wch[c]
                dotvec = jnp.where(lanes == j, jnp.sum(acc), dotvec)
            dot_v[pl.ds(ci * VL, VL)] = dotvec
            return 0

        lax.fori_loop(0, BPW // VL, chunk, 0)

    drain(semb_hbm, idxs_v, sbuf, ssems)
    dots(sbuf, 0, dots_v)
    drain(temb_hbm, idxt_v, tbuf, tsems)
    dots(tbuf, D, dott_v)

    b0 = w_v[1, pl.ds(0, VL)][0]
    for c in range(BPW // VL):
        sl = pl.ds(c * VL, VL)
        s = dots_v[sl] * recs_v[sl] + dott_v[sl] * rect_v[sl] + b0
        out_v[sl] = 1.0 / (1.0 + jnp.exp(-s))

    pltpu.sync_copy(out_v, out_hbm.at[wid])


_sc_call_cache = []


def _get_sc_call():
    if not _sc_call_cache:
        mesh = plsc.VectorSubcoreMesh(core_axis_name="c", subcore_axis_name="s",
                                      num_cores=NC, num_subcores=NS)
        _sc_call_cache.append(pl.kernel(
            _body,
            out_type=jax.ShapeDtypeStruct((NW, BPW), jnp.float32),
            mesh=mesh,
            compiler_params=pltpu.CompilerParams(needs_layout_passes=False,
                                                 use_tc_tiling_on_sc=False),
            scratch_types=[
                pltpu.VMEM((BPW, L), jnp.int32),
                pltpu.VMEM((BPW, L), jnp.int32),
                pltpu.VMEM((L, BPW), jnp.int32),
                pltpu.VMEM((L, BPW), jnp.int32),
                pltpu.VMEM((NBUF, BPW, D), jnp.float32),
                pltpu.VMEM((NBUF, BPW, D), jnp.float32),
                pltpu.VMEM((8, 128), jnp.float32),
                pltpu.VMEM((BPW,), jnp.float32),
                pltpu.VMEM((BPW,), jnp.float32),
                pltpu.VMEM((BPW,), jnp.float32),
                pltpu.VMEM((BPW,), jnp.float32),
                pltpu.VMEM((BPW,), jnp.float32),
                pltpu.SemaphoreType.DMA((NBUF,)),
                pltpu.SemaphoreType.DMA((NBUF,)),
            ],
        ))
    return _sc_call_cache[0]


@jax.jit
def kernel(src_tokens, trg_tokens, src_emb, trg_emb, W_out, b_out):
    par = (jnp.zeros((8, 128), jnp.float32)
           .at[0, :].set(W_out.reshape(-1))
           .at[1, 0].set(b_out[0]))
    out = _get_sc_call()(src_tokens.astype(jnp.int32),
                         trg_tokens.astype(jnp.int32),
                         src_emb, trg_emb, par)
    return out.reshape(B, 1)
